# baseline (speedup 1.0000x reference)
import functools

import jax
import jax.numpy as jnp
from jax import lax
from jax.experimental import pallas as pl
from jax.experimental.pallas import tpu as pltpu
from jax.experimental.pallas import tpu_sc as plsc

VOCAB = 1000000
D_MODEL = 128
BATCH = 4
SEQ_LEN = 8192

NC = 2
NS = 16
L = 16
NW = NC * NS

N_ROWS = BATCH * SEQ_LEN
ROWS_PER_W = N_ROWS // NW
CHUNK = 256
N_CHUNKS = ROWS_PER_W // CHUNK
VECS_PER_ROW = D_MODEL // L


def _body(idx_hbm, tok_hbm, pos_hbm, out_hbm, idx_v, rows_v, pos_v, sem):
    wid = lax.axis_index("s") * NC + lax.axis_index("c")
    base = wid * ROWS_PER_W
    s_base = lax.rem(base, SEQ_LEN)

    for c in range(N_CHUNKS):
        off = base + c * CHUNK
        s0 = s_base + c * CHUNK
        pltpu.sync_copy(idx_hbm.at[pl.ds(off, CHUNK)], idx_v)
        pltpu.async_copy(tok_hbm.at[idx_v], rows_v, sem).wait()
        pltpu.sync_copy(pos_hbm.at[pl.ds(s0, CHUNK)], pos_v)

        def add_row(r, _):
            for j in range(VECS_PER_ROW):
                sl = pl.ds(j * L, L)
                rows_v[r, sl] = rows_v[r, sl] + pos_v[r, sl]
            return _

        lax.fori_loop(0, CHUNK, add_row, 0)
        pltpu.sync_copy(rows_v, out_hbm.at[pl.ds(off, CHUNK)])


@jax.jit
def _embed(x_flat, token_table, pos_table):
    mesh = plsc.VectorSubcoreMesh(core_axis_name="c", subcore_axis_name="s")
    k = pl.kernel(
        _body,
        out_type=jax.ShapeDtypeStruct((N_ROWS, D_MODEL), jnp.float32),
        mesh=mesh,
        scratch_types=[
            pltpu.VMEM((CHUNK,), jnp.int32),
            pltpu.VMEM((CHUNK, D_MODEL), jnp.float32),
            pltpu.VMEM((CHUNK, D_MODEL), jnp.float32),
            pltpu.SemaphoreType.DMA,
        ],
    )
    return k(x_flat, token_table, pos_table)


def kernel(x, token_table, pos_table):
    x_flat = x.reshape(-1).astype(jnp.int32)
    out = _embed(x_flat, token_table, pos_table)
    return out.reshape(BATCH, SEQ_LEN, D_MODEL)


# speedup vs baseline: 1.1983x; 1.1983x over previous
import functools

import jax
import jax.numpy as jnp
from jax import lax
from jax.experimental import pallas as pl
from jax.experimental.pallas import tpu as pltpu
from jax.experimental.pallas import tpu_sc as plsc

VOCAB = 1000000
D_MODEL = 128
BATCH = 4
SEQ_LEN = 8192

NC = 2
NS = 16
L = 16
NW = NC * NS

N_ROWS = BATCH * SEQ_LEN
ROWS_PER_W = N_ROWS // NW
CHUNK = 256
N_CHUNKS = ROWS_PER_W // CHUNK
VECS_PER_ROW = D_MODEL // L


def _body(idx_hbm, tok_hbm, pos_hbm, out_hbm, idx_v, rows_v, pos_v, sem):
    wid = lax.axis_index("s") * NC + lax.axis_index("c")
    base = wid * ROWS_PER_W
    s_base = lax.rem(base, SEQ_LEN)

    for c in range(N_CHUNKS):
        off = base + c * CHUNK
        s0 = s_base + c * CHUNK
        pltpu.sync_copy(idx_hbm.at[pl.ds(off, CHUNK)], idx_v)
        pltpu.sync_copy(pos_hbm.at[pl.ds(s0, CHUNK)], rows_v)
        pltpu.async_copy(tok_hbm.at[idx_v], rows_v, sem, add=True).wait()
        pltpu.sync_copy(rows_v, out_hbm.at[pl.ds(off, CHUNK)])


@jax.jit
def _embed(x_flat, token_table, pos_table):
    mesh = plsc.VectorSubcoreMesh(core_axis_name="c", subcore_axis_name="s")
    k = pl.kernel(
        _body,
        out_type=jax.ShapeDtypeStruct((N_ROWS, D_MODEL), jnp.float32),
        mesh=mesh,
        scratch_types=[
            pltpu.VMEM((CHUNK,), jnp.int32),
            pltpu.VMEM((CHUNK, D_MODEL), jnp.float32),
            pltpu.VMEM((CHUNK, D_MODEL), jnp.float32),
            pltpu.SemaphoreType.DMA,
        ],
    )
    return k(x_flat, token_table, pos_table)


def kernel(x, token_table, pos_table):
    x_flat = x.reshape(-1).astype(jnp.int32)
    out = _embed(x_flat, token_table, pos_table)
    return out.reshape(BATCH, SEQ_LEN, D_MODEL)


# speedup vs baseline: 1.3962x; 1.1651x over previous
import jax
import jax.numpy as jnp
from jax import lax
from jax.experimental import pallas as pl
from jax.experimental.pallas import tpu as pltpu
from jax.experimental.pallas import tpu_sc as plsc

VOCAB = 1000000
D_MODEL = 128
BATCH = 4
SEQ_LEN = 8192

NC = 2
NS = 16
NW = NC * NS

N_ROWS = BATCH * SEQ_LEN
S_PER_W = SEQ_LEN // NW
NBUF = 2


def _body(idx_hbm, tok_hbm, pos_hbm, out_hbm,
          idx0, idx1, idx2, idx3, pos_sh, acc_v, sem_g, sem_o):
    idx_bufs = [idx0, idx1, idx2, idx3]
    wid = lax.axis_index("s") * NC + lax.axis_index("c")
    sid = lax.axis_index("s")
    s0 = wid * S_PER_W
    my_pos = pos_sh.at[pl.ds(sid * S_PER_W, S_PER_W)]

    pltpu.sync_copy(pos_hbm.at[pl.ds(s0, S_PER_W)], my_pos)
    for b in range(BATCH):
        pltpu.sync_copy(idx_hbm.at[pl.ds(b * SEQ_LEN + s0, S_PER_W)],
                        idx_bufs[b])

    def prep(b):
        p = b % NBUF
        pltpu.sync_copy(my_pos, acc_v.at[p])
        pltpu.async_copy(tok_hbm.at[idx_bufs[b]], acc_v.at[p], sem_g.at[p],
                         add=True)

    prep(0)
    prep(1)
    for b in range(BATCH):
        p = b % NBUF
        pltpu.make_async_copy(tok_hbm.at[idx_bufs[b]], acc_v.at[p],
                              sem_g.at[p]).wait()
        pltpu.async_copy(acc_v.at[p], out_hbm.at[pl.ds(b * SEQ_LEN + s0,
                                                       S_PER_W)], sem_o.at[p])
        if b + NBUF < BATCH:
            pltpu.make_async_copy(acc_v.at[p],
                                  out_hbm.at[pl.ds(b * SEQ_LEN + s0, S_PER_W)],
                                  sem_o.at[p]).wait()
            prep(b + NBUF)
    for b in range(BATCH - NBUF, BATCH):
        p = b % NBUF
        pltpu.make_async_copy(acc_v.at[p],
                              out_hbm.at[pl.ds(b * SEQ_LEN + s0, S_PER_W)],
                              sem_o.at[p]).wait()


@jax.jit
def _embed(x_flat, token_table, pos_table):
    mesh = plsc.VectorSubcoreMesh(core_axis_name="c", subcore_axis_name="s")
    k = pl.kernel(
        _body,
        out_type=jax.ShapeDtypeStruct((N_ROWS, D_MODEL), jnp.float32),
        mesh=mesh,
        scratch_types=[
            pltpu.VMEM((S_PER_W,), jnp.int32),
            pltpu.VMEM((S_PER_W,), jnp.int32),
            pltpu.VMEM((S_PER_W,), jnp.int32),
            pltpu.VMEM((S_PER_W,), jnp.int32),
            pltpu.VMEM_SHARED((NS * S_PER_W, D_MODEL), jnp.float32),
            pltpu.VMEM((NBUF, S_PER_W, D_MODEL), jnp.float32),
            pltpu.SemaphoreType.DMA((NBUF,)),
            pltpu.SemaphoreType.DMA((NBUF,)),
        ],
    )
    return k(x_flat, token_table, pos_table)


def kernel(x, token_table, pos_table):
    x_flat = x.reshape(-1).astype(jnp.int32)
    out = _embed(x_flat, token_table, pos_table)
    return out.reshape(BATCH, SEQ_LEN, D_MODEL)


# speedup vs baseline: 1.4017x; 1.0039x over previous
import jax
import jax.numpy as jnp
from jax import lax
from jax.experimental import pallas as pl
from jax.experimental.pallas import tpu as pltpu
from jax.experimental.pallas import tpu_sc as plsc

VOCAB = 1000000
D_MODEL = 128
BATCH = 4
SEQ_LEN = 8192

NC = 2
NS = 16
NW = NC * NS

S_PER_W = SEQ_LEN // NW
NBUF = 2


def _body(idx_hbm, tok_hbm, pos_hbm, out_hbm,
          idx0, idx1, idx2, idx3, pos_sh, acc_v, sem_g, sem_o):
    idx_bufs = [idx0, idx1, idx2, idx3]
    wid = lax.axis_index("s") * NC + lax.axis_index("c")
    sid = lax.axis_index("s")
    s0 = wid * S_PER_W
    my_pos = pos_sh.at[pl.ds(sid * S_PER_W, S_PER_W)]

    pltpu.sync_copy(pos_hbm.at[pl.ds(s0, S_PER_W)], my_pos)
    for b in range(BATCH):
        pltpu.sync_copy(idx_hbm.at[b, pl.ds(s0, S_PER_W)], idx_bufs[b])

    def prep(b):
        p = b % NBUF
        pltpu.sync_copy(my_pos, acc_v.at[p])
        pltpu.async_copy(tok_hbm.at[idx_bufs[b]], acc_v.at[p], sem_g.at[p],
                         add=True)

    prep(0)
    prep(1)
    for b in range(BATCH):
        p = b % NBUF
        pltpu.make_async_copy(tok_hbm.at[idx_bufs[b]], acc_v.at[p],
                              sem_g.at[p]).wait()
        pltpu.async_copy(acc_v.at[p], out_hbm.at[b, pl.ds(s0, S_PER_W)],
                         sem_o.at[p])
        if b + NBUF < BATCH:
            pltpu.make_async_copy(acc_v.at[p],
                                  out_hbm.at[b, pl.ds(s0, S_PER_W)],
                                  sem_o.at[p]).wait()
            prep(b + NBUF)
    for b in range(BATCH - NBUF, BATCH):
        p = b % NBUF
        pltpu.make_async_copy(acc_v.at[p],
                              out_hbm.at[b, pl.ds(s0, S_PER_W)],
                              sem_o.at[p]).wait()


@jax.jit
def _embed(x, token_table, pos_table):
    mesh = plsc.VectorSubcoreMesh(core_axis_name="c", subcore_axis_name="s")
    k = pl.kernel(
        _body,
        out_type=jax.ShapeDtypeStruct((BATCH, SEQ_LEN, D_MODEL), jnp.float32),
        mesh=mesh,
        scratch_types=[
            pltpu.VMEM((S_PER_W,), jnp.int32),
            pltpu.VMEM((S_PER_W,), jnp.int32),
            pltpu.VMEM((S_PER_W,), jnp.int32),
            pltpu.VMEM((S_PER_W,), jnp.int32),
            pltpu.VMEM_SHARED((NS * S_PER_W, D_MODEL), jnp.float32),
            pltpu.VMEM((NBUF, S_PER_W, D_MODEL), jnp.float32),
            pltpu.SemaphoreType.DMA((NBUF,)),
            pltpu.SemaphoreType.DMA((NBUF,)),
        ],
    )
    return k(x, token_table, pos_table)


def kernel(x, token_table, pos_table):
    return _embed(x.astype(jnp.int32), token_table, pos_table)


# speedup vs baseline: 1.4594x; 1.0412x over previous
import jax
import jax.numpy as jnp
from jax import lax
from jax.experimental import pallas as pl
from jax.experimental.pallas import tpu as pltpu
from jax.experimental.pallas import tpu_sc as plsc

VOCAB = 1000000
D_MODEL = 128
BATCH = 4
SEQ_LEN = 8192

NC = 2
NS = 16
NW = NC * NS

S_PER_W = SEQ_LEN // NW
CH = 128
HALVES = S_PER_W // CH
NCH = BATCH * HALVES
NBUF = 4


def _body(idx_hbm, tok_hbm, pos_hbm, out_hbm,
          i0, i1, i2, i3, i4, i5, i6, i7, pos_sh, acc_v,
          sem_i, sem_p, sem_c, sem_g, sem_o):
    idx_bufs = [i0, i1, i2, i3, i4, i5, i6, i7]
    wid = lax.axis_index("s") * NC + lax.axis_index("c")
    sid = lax.axis_index("s")
    s0 = wid * S_PER_W

    def bh(t):
        return t // HALVES, (t % HALVES) * CH

    my_pos = pos_sh.at[pl.ds(sid * S_PER_W, S_PER_W)]
    pltpu.async_copy(pos_hbm.at[pl.ds(s0, S_PER_W)], my_pos, sem_p)
    for t in range(NCH):
        b, h = bh(t)
        pltpu.async_copy(idx_hbm.at[b, pl.ds(s0 + h, CH)], idx_bufs[t], sem_i)
    pltpu.make_async_copy(pos_hbm.at[pl.ds(s0, S_PER_W)], my_pos, sem_p).wait()
    for t in range(NCH):
        b, h = bh(t)
        pltpu.make_async_copy(idx_hbm.at[b, pl.ds(s0 + h, CH)],
                              idx_bufs[t], sem_i).wait()

    def spos(t):
        return pos_sh.at[pl.ds(sid * S_PER_W + bh(t)[1], CH)]

    def oslice(t):
        b, h = bh(t)
        return out_hbm.at[b, pl.ds(s0 + h, CH)]

    def issue_posc(t):
        pltpu.async_copy(spos(t), acc_v.at[t % NBUF], sem_c.at[t % NBUF])

    def wait_posc(t):
        pltpu.make_async_copy(spos(t), acc_v.at[t % NBUF],
                              sem_c.at[t % NBUF]).wait()

    def issue_gather(t):
        pltpu.async_copy(tok_hbm.at[idx_bufs[t]], acc_v.at[t % NBUF],
                         sem_g.at[t % NBUF], add=True)

    def wait_gather(t):
        pltpu.make_async_copy(tok_hbm.at[idx_bufs[t]], acc_v.at[t % NBUF],
                              sem_g.at[t % NBUF]).wait()

    def issue_out(t):
        pltpu.async_copy(acc_v.at[t % NBUF], oslice(t), sem_o.at[t % NBUF])

    def wait_out(t):
        pltpu.make_async_copy(acc_v.at[t % NBUF], oslice(t),
                              sem_o.at[t % NBUF]).wait()

    for t in range(NBUF):
        issue_posc(t)
    for t in range(NCH):
        wait_posc(t)
        issue_gather(t)
        if t >= 1:
            wait_gather(t - 1)
            issue_out(t - 1)
        nxt = t + 1
        if NBUF <= nxt < NCH:
            wait_out(nxt - NBUF)
            issue_posc(nxt)
    wait_gather(NCH - 1)
    issue_out(NCH - 1)
    for t in range(NCH - NBUF, NCH):
        wait_out(t)


@jax.jit
def _embed(x, token_table, pos_table):
    mesh = plsc.VectorSubcoreMesh(core_axis_name="c", subcore_axis_name="s")
    k = pl.kernel(
        _body,
        out_type=jax.ShapeDtypeStruct((BATCH, SEQ_LEN, D_MODEL), jnp.float32),
        mesh=mesh,
        scratch_types=(
            [pltpu.VMEM((CH,), jnp.int32) for _ in range(NCH)]
            + [
                pltpu.VMEM_SHARED((NS * S_PER_W, D_MODEL), jnp.float32),
                pltpu.VMEM((NBUF, CH, D_MODEL), jnp.float32),
                pltpu.SemaphoreType.DMA,
                pltpu.SemaphoreType.DMA,
                pltpu.SemaphoreType.DMA((NBUF,)),
                pltpu.SemaphoreType.DMA((NBUF,)),
                pltpu.SemaphoreType.DMA((NBUF,)),
            ]
        ),
    )
    return k(x, token_table, pos_table)


def kernel(x, token_table, pos_table):
    return _embed(x.astype(jnp.int32), token_table, pos_table)


# speedup vs baseline: 1.4670x; 1.0053x over previous
import jax
import jax.numpy as jnp
from jax import lax
from jax.experimental import pallas as pl
from jax.experimental.pallas import tpu as pltpu
from jax.experimental.pallas import tpu_sc as plsc

VOCAB = 1000000
D_MODEL = 128
BATCH = 4
SEQ_LEN = 8192

NC = 2
NS = 16
NW = NC * NS

S_PER_W = SEQ_LEN // NW
CH = 128
HALVES = S_PER_W // CH
NCH = BATCH * HALVES
NBUF = 4


def _body(idx_hbm, tok_hbm, pos_hbm, out_hbm,
          i0, i1, i2, i3, i4, i5, i6, i7, pos_sh, acc_v,
          sem_i, sem_p, sem_c, sem_g, sem_o):
    idx_bufs = [i0, i1, i2, i3, i4, i5, i6, i7]
    wid = lax.axis_index("s") * NC + lax.axis_index("c")
    sid = lax.axis_index("s")
    s0 = wid * S_PER_W

    def bh(t):
        return t // HALVES, (t % HALVES) * CH

    my_pos = pos_sh.at[pl.ds(sid * S_PER_W, S_PER_W)]
    pltpu.async_copy(pos_hbm.at[pl.ds(s0, S_PER_W)], my_pos, sem_p)
    for t in range(NCH):
        b, h = bh(t)
        pltpu.async_copy(idx_hbm.at[b, pl.ds(s0 + h, CH)], idx_bufs[t], sem_i)
    def spos(t):
        return pos_sh.at[pl.ds(sid * S_PER_W + bh(t)[1], CH)]

    def oslice(t):
        b, h = bh(t)
        return out_hbm.at[b, pl.ds(s0 + h, CH)]

    def issue_posc(t):
        pltpu.async_copy(spos(t), acc_v.at[t % NBUF], sem_c.at[t % NBUF])

    def wait_posc(t):
        pltpu.make_async_copy(spos(t), acc_v.at[t % NBUF],
                              sem_c.at[t % NBUF]).wait()

    def issue_gather(t):
        pltpu.async_copy(tok_hbm.at[idx_bufs[t]], acc_v.at[t % NBUF],
                         sem_g.at[t % NBUF], add=True)

    def wait_gather(t):
        pltpu.make_async_copy(tok_hbm.at[idx_bufs[t]], acc_v.at[t % NBUF],
                              sem_g.at[t % NBUF]).wait()

    def issue_out(t):
        pltpu.async_copy(acc_v.at[t % NBUF], oslice(t), sem_o.at[t % NBUF])

    def wait_out(t):
        pltpu.make_async_copy(acc_v.at[t % NBUF], oslice(t),
                              sem_o.at[t % NBUF]).wait()

    pltpu.make_async_copy(pos_hbm.at[pl.ds(s0, S_PER_W)], my_pos, sem_p).wait()
    for t in range(NBUF):
        issue_posc(t)
    for t in range(NCH):
        b, h = bh(t)
        pltpu.make_async_copy(idx_hbm.at[b, pl.ds(s0 + h, CH)],
                              idx_bufs[t], sem_i).wait()
    for t in range(NCH):
        wait_posc(t)
        issue_gather(t)
        if t >= 1:
            wait_gather(t - 1)
            issue_out(t - 1)
        nxt = t + 1
        if NBUF <= nxt < NCH:
            wait_out(nxt - NBUF)
            issue_posc(nxt)
    wait_gather(NCH - 1)
    issue_out(NCH - 1)
    for t in range(NCH - NBUF, NCH):
        wait_out(t)


@jax.jit
def _embed(x, token_table, pos_table):
    mesh = plsc.VectorSubcoreMesh(core_axis_name="c", subcore_axis_name="s")
    k = pl.kernel(
        _body,
        out_type=jax.ShapeDtypeStruct((BATCH, SEQ_LEN, D_MODEL), jnp.float32),
        mesh=mesh,
        scratch_types=(
            [pltpu.VMEM((CH,), jnp.int32) for _ in range(NCH)]
            + [
                pltpu.VMEM_SHARED((NS * S_PER_W, D_MODEL), jnp.float32),
                pltpu.VMEM((NBUF, CH, D_MODEL), jnp.float32),
                pltpu.SemaphoreType.DMA,
                pltpu.SemaphoreType.DMA,
                pltpu.SemaphoreType.DMA((NBUF,)),
                pltpu.SemaphoreType.DMA((NBUF,)),
                pltpu.SemaphoreType.DMA((NBUF,)),
            ]
        ),
    )
    return k(x, token_table, pos_table)


def kernel(x, token_table, pos_table):
    return _embed(x.astype(jnp.int32), token_table, pos_table)
